# Initial kernel scaffold; baseline (speedup 1.0000x reference)
import jax
import jax.numpy as jnp
from jax.experimental import pallas as pl


def kernel(x, node_embedding, target_node):
    raise NotImplementedError("write your pallas kernel here")


# speedup vs baseline: 7.7677x; 7.7677x over previous
import functools

import jax
import jax.numpy as jnp
from jax import lax
from jax.experimental import pallas as pl
from jax.experimental.pallas import tpu as pltpu
from jax.experimental.pallas import tpu_sc as plsc

B = 8
TOPK = 16

NC = 2
NS = 16
NW = NC * NS


def _select_body(emb_ref, tgt_ref, idx_ref):
    M, C = emb_ref.shape[1], emb_ref.shape[2]
    emb = emb_ref[...]
    t = tgt_ref[0, 0, :]
    j_iota = lax.broadcasted_iota(jnp.int32, (B, M), 1)
    oh = (j_iota == t[:, None]).astype(jnp.float32)
    e_t = jnp.sum(emb * oh[:, :, None], axis=1)
    ns = jnp.sqrt(jnp.sum(emb * emb, axis=-1))
    n_t = jnp.sum(ns * oh, axis=1)
    emb_b = emb.astype(jnp.bfloat16)
    et_b = e_t.astype(jnp.bfloat16)
    rows = []
    for b in range(B):
        rows.append(lax.dot_general(
            et_b[b].reshape(1, C), emb_b[b],
            (((1,), (1,)), ((), ())),
            preferred_element_type=jnp.float32))
    d = jnp.concatenate(rows, axis=0)
    s = d / (n_t[:, None] * ns)
    cols = []
    for _ in range(TOPK):
        m = jnp.max(s, axis=-1, keepdims=True)
        jk = jnp.min(jnp.where(s == m, j_iota, M), axis=-1)
        cols.append(jk.reshape(B, 1))
        s = jnp.where(j_iota == jk[:, None], -jnp.inf, s)
    idx_local = jnp.concatenate(cols, axis=1)
    row = pl.program_id(0) * B + lax.broadcasted_iota(jnp.int32, (B, TOPK), 0)
    idx_ref[0] = row * M + idx_local


def _select(emb, tgt):
    N, M, C = emb.shape
    return pl.pallas_call(
        _select_body,
        grid=(N // B,),
        in_specs=[
            pl.BlockSpec((B, M, C), lambda i: (i, 0, 0)),
            pl.BlockSpec((1, 1, B), lambda i: (i, 0, 0)),
        ],
        out_specs=pl.BlockSpec((1, B, TOPK), lambda i: (i, 0, 0)),
        out_shape=jax.ShapeDtypeStruct((N // B, B, TOPK), jnp.int32),
    )(emb, tgt.reshape(N // B, 1, B))


def _gather(x_flat, emb_flat, idx2d):
    rows_total, lanes = idx2d.shape
    rpw = rows_total // NW
    C = x_flat.shape[1]
    out_sds = jax.ShapeDtypeStruct((rows_total, lanes, C), jnp.float32)
    mesh = plsc.VectorSubcoreMesh(core_axis_name="c", subcore_axis_name="s")

    @functools.partial(
        pl.kernel, mesh=mesh,
        compiler_params=pltpu.CompilerParams(use_tc_tiling_on_sc=False),
        out_type=(out_sds, out_sds),
        scratch_types=[
            pltpu.VMEM((rpw, lanes), jnp.int32),
            pltpu.VMEM((rpw, lanes, C), jnp.float32),
            pltpu.VMEM((rpw, lanes, C), jnp.float32),
            pltpu.SemaphoreType.DMA,
        ],
    )
    def k(x_hbm, e_hbm, idx_hbm, xo_hbm, eo_hbm, idx_v, rx_v, re_v, sem):
        wid = lax.axis_index("s") * NC + lax.axis_index("c")
        base = wid * rpw
        pltpu.sync_copy(idx_hbm.at[pl.ds(base, rpw)], idx_v)
        for r in range(rpw):
            a = pltpu.async_copy(x_hbm.at[idx_v.at[r]], rx_v.at[r], sem)
            b = pltpu.async_copy(e_hbm.at[idx_v.at[r]], re_v.at[r], sem)
            a.wait()
            b.wait()
        pltpu.sync_copy(rx_v, xo_hbm.at[pl.ds(base, rpw)])
        pltpu.sync_copy(re_v, eo_hbm.at[pl.ds(base, rpw)])

    return k(x_flat, emb_flat, idx2d)


def kernel(x, node_embedding, target_node):
    N, M, C = x.shape
    emb = lax.stop_gradient(node_embedding)
    t = target_node.astype(jnp.int32)
    flat_idx = _select(emb, t).reshape(N * TOPK)
    idx2d = flat_idx.reshape(N * TOPK // 128, 128)
    x_sel, emb_sel = _gather(
        x.reshape(N * M, C), node_embedding.reshape(N * M, C), idx2d)
    return (x_sel.reshape(N, TOPK, C), emb_sel.reshape(N, TOPK, C))


# speedup vs baseline: 18.4926x; 2.3807x over previous
import functools

import jax
import jax.numpy as jnp
from jax import lax
from jax.experimental import pallas as pl
from jax.experimental.pallas import tpu as pltpu
from jax.experimental.pallas import tpu_sc as plsc

B = 128
TOPK = 16

NC = 2
NS = 16
NW = NC * NS


def _select_body(tgt_ref, emb_ref, idx_ref, d_ref, et_ref):
    M, C = emb_ref.shape[1], emb_ref.shape[2]
    emb = emb_ref[...]
    ns = jnp.sqrt(jnp.sum(emb * emb, axis=-1))
    emb_b = emb.astype(jnp.bfloat16)
    base = pl.program_id(0) * B
    for b in range(B):
        t_b = tgt_ref[base + b]
        er = emb_ref[b, t_b, :]
        et_ref[pl.ds(b, 1), :] = er.reshape(1, C)
        d_ref[pl.ds(b, 1), :] = lax.dot_general(
            er.reshape(1, C).astype(jnp.bfloat16), emb_b[b],
            (((1,), (1,)), ((), ())),
            preferred_element_type=jnp.float32)
    et = et_ref[...]
    n_t = jnp.sqrt(jnp.sum(et * et, axis=-1))
    s = d_ref[...] / (n_t[:, None] * ns)
    j_iota = lax.broadcasted_iota(jnp.int32, (B, M), 1)
    cols = []
    for _ in range(TOPK):
        m = jnp.max(s, axis=-1, keepdims=True)
        jk = jnp.min(jnp.where(s == m, j_iota, M), axis=-1)
        cols.append(jk.reshape(B, 1))
        s = jnp.where(j_iota == jk[:, None], -jnp.inf, s)
    idx_local = jnp.concatenate(cols, axis=1)
    row = base + lax.broadcasted_iota(jnp.int32, (B, TOPK), 0)
    idx_ref[0] = row * M + idx_local


def _select(emb, tgt):
    N, M, C = emb.shape
    return pl.pallas_call(
        _select_body,
        grid=(N // B,),
        in_specs=[
            pl.BlockSpec(memory_space=pltpu.SMEM),
            pl.BlockSpec((B, M, C), lambda i: (i, 0, 0)),
        ],
        out_specs=pl.BlockSpec((1, B, TOPK), lambda i: (i, 0, 0)),
        out_shape=jax.ShapeDtypeStruct((N // B, B, TOPK), jnp.int32),
        scratch_shapes=[
            pltpu.VMEM((B, M), jnp.float32),
            pltpu.VMEM((B, C), jnp.float32),
        ],
    )(tgt, emb)


def _gather(x_tiles, emb_tiles, idx2d, C):
    rows_total, lanes = idx2d.shape
    rpw = rows_total // NW
    bpw = rpw * lanes
    pack = lanes // C
    out_sds = jax.ShapeDtypeStruct((rows_total * lanes // pack, lanes),
                                   jnp.float32)
    mesh = plsc.VectorSubcoreMesh(core_axis_name="c", subcore_axis_name="s")
    lane_iota = lambda: lax.iota(jnp.int32, 16)

    @functools.partial(
        pl.kernel, mesh=mesh,
        compiler_params=pltpu.CompilerParams(needs_layout_passes=False),
        out_type=(out_sds, out_sds),
        scratch_types=[
            pltpu.VMEM((rpw, lanes), jnp.int32),
            pltpu.VMEM((rpw, lanes), jnp.int32),
            pltpu.VMEM((lanes, lanes), jnp.float32),
            pltpu.VMEM((bpw // pack, lanes), jnp.float32),
            pltpu.VMEM((bpw // pack, lanes), jnp.float32),
            pltpu.SemaphoreType.DMA,
        ],
    )
    def k(xt_hbm, et_hbm, idx_hbm, xo_hbm, eo_hbm,
          idx_v, tile_v, rows_v, ox_v, oe_v, sem):
        wid = lax.axis_index("s") * NC + lax.axis_index("c")
        base = wid * rpw
        pltpu.sync_copy(idx_hbm.at[pl.ds(base, rpw)], idx_v)
        for r in range(rpw):
            for kk in range(lanes // 16):
                sl = pl.ds(kk * 16, 16)
                tile_v[r, sl] = lax.shift_right_logical(idx_v[r, sl], 2)

        def compact(tab_hbm, out_v):
            for r in range(rpw):
                pltpu.async_copy(
                    tab_hbm.at[tile_v.at[r]], rows_v, sem).wait()

                def body(i, _):
                    sp = lambda v: lax.broadcast_in_dim(v, (16,), ())
                    fi = plsc.load_gather(idx_v, [sp(r), sp(i)])
                    rem = lax.bitwise_and(fi, pack - 1)
                    prow = r * (lanes // pack) + lax.shift_right_logical(i, 2)
                    cbase = lax.bitwise_and(i, pack - 1) * C
                    for half in range(C // 16):
                        col = rem * C + half * 16 + lane_iota()
                        v = plsc.load_gather(rows_v, [sp(i), col])
                        plsc.store_scatter(
                            out_v,
                            [sp(prow), sp(cbase) + half * 16 + lane_iota()],
                            v)
                    return 0

                lax.fori_loop(0, lanes, body, 0)

        compact(xt_hbm, ox_v)
        compact(et_hbm, oe_v)
        opw = bpw // pack
        pltpu.sync_copy(ox_v, xo_hbm.at[pl.ds(wid * opw, opw)])
        pltpu.sync_copy(oe_v, eo_hbm.at[pl.ds(wid * opw, opw)])

    return k(x_tiles, emb_tiles, idx2d)


def kernel(x, node_embedding, target_node):
    N, M, C = x.shape
    emb = lax.stop_gradient(node_embedding)
    t = target_node.astype(jnp.int32)
    flat_idx = _select(emb, t).reshape(N * TOPK)
    idx2d = flat_idx.reshape(N * TOPK // 128, 128)
    tiles_per_row = 128 // C
    x_sel, emb_sel = _gather(
        x.reshape(N * M // tiles_per_row, 128),
        node_embedding.reshape(N * M // tiles_per_row, 128),
        idx2d, C)
    return (x_sel.reshape(N, TOPK, C), emb_sel.reshape(N, TOPK, C))


# speedup vs baseline: 19.8867x; 1.0754x over previous
import functools

import jax
import jax.numpy as jnp
from jax import lax
from jax.experimental import pallas as pl
from jax.experimental.pallas import tpu as pltpu
from jax.experimental.pallas import tpu_sc as plsc

B = 128
TOPK = 16

NC = 2
NS = 16
NW = NC * NS


def _select_body(tgt_ref, emb_ref, idx_ref, d_ref, et_ref):
    M, C = emb_ref.shape[1], emb_ref.shape[2]
    emb = emb_ref[...]
    ns = jnp.sqrt(jnp.sum(emb * emb, axis=-1))
    emb_b = emb.astype(jnp.bfloat16)
    base = pl.program_id(0) * B
    for b in range(B):
        t_b = tgt_ref[base + b]
        er = emb_ref[b, t_b, :]
        et_ref[pl.ds(b, 1), :] = er.reshape(1, C)
        d_ref[pl.ds(b, 1), :] = lax.dot_general(
            er.reshape(1, C).astype(jnp.bfloat16), emb_b[b],
            (((1,), (1,)), ((), ())),
            preferred_element_type=jnp.float32)
    et = et_ref[...]
    n_t = jnp.sqrt(jnp.sum(et * et, axis=-1))
    s = d_ref[...] / (n_t[:, None] * ns)
    j_iota = lax.broadcasted_iota(jnp.int32, (B, M), 1)
    cols = []
    for _ in range(TOPK):
        m = jnp.max(s, axis=-1, keepdims=True)
        jk = jnp.min(jnp.where(s == m, j_iota, M), axis=-1)
        cols.append(jk.reshape(B, 1))
        s = jnp.where(j_iota == jk[:, None], -jnp.inf, s)
    idx_local = jnp.concatenate(cols, axis=1)
    row = base + lax.broadcasted_iota(jnp.int32, (B, TOPK), 0)
    idx_ref[0] = row * M + idx_local


def _select(emb, tgt):
    N, M, C = emb.shape
    return pl.pallas_call(
        _select_body,
        grid=(N // B,),
        in_specs=[
            pl.BlockSpec(memory_space=pltpu.SMEM),
            pl.BlockSpec((B, M, C), lambda i: (i, 0, 0)),
        ],
        out_specs=pl.BlockSpec((1, B, TOPK), lambda i: (i, 0, 0)),
        out_shape=jax.ShapeDtypeStruct((N // B, B, TOPK), jnp.int32),
        scratch_shapes=[
            pltpu.VMEM((B, M), jnp.float32),
            pltpu.VMEM((B, C), jnp.float32),
        ],
    )(tgt, emb)


def _gather(x_flat, emb_flat, idx2d):
    rows_total, lanes = idx2d.shape
    rpw = rows_total // NW
    bpw = rpw * lanes
    C = x_flat.shape[1]
    out_sds = jax.ShapeDtypeStruct((rows_total * lanes, C), jnp.float32)
    mesh = plsc.VectorSubcoreMesh(core_axis_name="c", subcore_axis_name="s")

    @functools.partial(
        pl.kernel, mesh=mesh,
        compiler_params=pltpu.CompilerParams(use_tc_tiling_on_sc=False),
        out_type=(out_sds, out_sds),
        scratch_types=[
            pltpu.VMEM((rpw, lanes), jnp.int32),
            pltpu.VMEM((bpw, C), jnp.float32),
            pltpu.VMEM((bpw, C), jnp.float32),
            pltpu.SemaphoreType.DMA,
        ],
    )
    def k(xt_hbm, et_hbm, idx_hbm, xo_hbm, eo_hbm, idx_v, ox_v, oe_v, sem):
        wid = lax.axis_index("s") * NC + lax.axis_index("c")
        base = wid * rpw
        pltpu.sync_copy(idx_hbm.at[pl.ds(base, rpw)], idx_v)
        hs = []
        for r in range(rpw):
            sl = pl.ds(r * lanes, lanes)
            hs.append(pltpu.async_copy(
                xt_hbm.at[idx_v.at[r]], ox_v.at[sl], sem))
            hs.append(pltpu.async_copy(
                et_hbm.at[idx_v.at[r]], oe_v.at[sl], sem))
        for h in hs:
            h.wait()
        pltpu.sync_copy(ox_v, xo_hbm.at[pl.ds(wid * bpw, bpw)])
        pltpu.sync_copy(oe_v, eo_hbm.at[pl.ds(wid * bpw, bpw)])

    return k(x_flat, emb_flat, idx2d)


def kernel(x, node_embedding, target_node):
    N, M, C = x.shape
    emb = lax.stop_gradient(node_embedding)
    t = target_node.astype(jnp.int32)
    flat_idx = _select(emb, t).reshape(N * TOPK)
    idx2d = flat_idx.reshape(N * TOPK // 128, 128)
    x_sel, emb_sel = _gather(
        x.reshape(N * M, C), node_embedding.reshape(N * M, C), idx2d)
    return (x_sel.reshape(N, TOPK, C), emb_sel.reshape(N, TOPK, C))
